# baseline (speedup 1.0000x reference)
import jax
import jax.numpy as jnp
from jax.experimental import pallas as pl
from jax.experimental.pallas import tpu as pltpu

DICT_LEN = 128
SEQ_LEN, BZ, D_MODEL, HIDDEN = 8192, 4, 768, 256
TILE_S = 512


def _switcher_kernel(gid_ref, x_ref, w1_ref, b1_ref, w2_ref, b2_ref, o_ref):
    for b in range(BZ):
        g = gid_ref[b]
        x = x_ref[:, b * D_MODEL:(b + 1) * D_MODEL]
        h = jnp.dot(x, w1_ref[g], preferred_element_type=jnp.float32)
        h = h + b1_ref[g]
        h = 0.5 * h * (1.0 + jax.lax.erf(h * 0.7071067811865476))
        out = jnp.dot(h, w2_ref[g], preferred_element_type=jnp.float32)
        o_ref[:, b * D_MODEL:(b + 1) * D_MODEL] = out + b2_ref[g]


def kernel(x, lang_ids, w1_0, b1_0, w2_0, b2_0, w1_1, b1_1, w2_1, b2_1):
    gid = (DICT_LEN - 1 - lang_ids.astype(jnp.int32) <= 3).astype(jnp.int32)
    w1 = jnp.stack([w1_0.T, w1_1.T])
    b1 = jnp.stack([b1_0, b1_1])[:, None, :]
    w2 = jnp.stack([w2_0.T, w2_1.T])
    b2 = jnp.stack([b2_0, b2_1])[:, None, :]
    xf = x.reshape(SEQ_LEN, BZ * D_MODEL)
    grid = (SEQ_LEN // TILE_S,)
    out = pl.pallas_call(
        _switcher_kernel,
        grid_spec=pltpu.PrefetchScalarGridSpec(
            num_scalar_prefetch=1,
            grid=grid,
            in_specs=[
                pl.BlockSpec((TILE_S, BZ * D_MODEL), lambda s, g: (s, 0)),
                pl.BlockSpec((2, D_MODEL, HIDDEN), lambda s, g: (0, 0, 0)),
                pl.BlockSpec((2, 1, HIDDEN), lambda s, g: (0, 0, 0)),
                pl.BlockSpec((2, HIDDEN, D_MODEL), lambda s, g: (0, 0, 0)),
                pl.BlockSpec((2, 1, D_MODEL), lambda s, g: (0, 0, 0)),
            ],
            out_specs=pl.BlockSpec((TILE_S, BZ * D_MODEL), lambda s, g: (s, 0)),
        ),
        out_shape=jax.ShapeDtypeStruct(xf.shape, xf.dtype),
        compiler_params=pltpu.CompilerParams(
            dimension_semantics=("arbitrary",),
        ),
    )(gid, xf, w1, b1, w2, b2)
    return out.reshape(SEQ_LEN, BZ, D_MODEL)


# speedup vs baseline: 2.8548x; 2.8548x over previous
import jax
import jax.numpy as jnp
from jax.experimental import pallas as pl
from jax.experimental.pallas import tpu as pltpu

DICT_LEN = 128
SEQ_LEN, BZ, D_MODEL, HIDDEN = 8192, 4, 768, 256
TILE_S = 512


def _switcher_kernel(gid_ref, x_ref, w1_ref, b1_ref, w2_ref, b2_ref, o_ref):
    for b in range(BZ):
        g = gid_ref[b]
        x = x_ref[:, b, :]
        h = jnp.dot(x, w1_ref[g], preferred_element_type=jnp.float32)
        h = h + b1_ref[g]
        h = 0.5 * h * (1.0 + jax.lax.erf(h * 0.7071067811865476))
        out = jnp.dot(h, w2_ref[g], preferred_element_type=jnp.float32)
        o_ref[:, b, :] = out + b2_ref[g]


def kernel(x, lang_ids, w1_0, b1_0, w2_0, b2_0, w1_1, b1_1, w2_1, b2_1):
    gid = (DICT_LEN - 1 - lang_ids.astype(jnp.int32) <= 3).astype(jnp.int32)
    w1 = jnp.stack([w1_0.T, w1_1.T])
    b1 = jnp.stack([b1_0, b1_1])[:, None, :]
    w2 = jnp.stack([w2_0.T, w2_1.T])
    b2 = jnp.stack([b2_0, b2_1])[:, None, :]
    grid = (SEQ_LEN // TILE_S,)
    return pl.pallas_call(
        _switcher_kernel,
        grid_spec=pltpu.PrefetchScalarGridSpec(
            num_scalar_prefetch=1,
            grid=grid,
            in_specs=[
                pl.BlockSpec((TILE_S, BZ, D_MODEL), lambda s, g: (s, 0, 0)),
                pl.BlockSpec((2, D_MODEL, HIDDEN), lambda s, g: (0, 0, 0)),
                pl.BlockSpec((2, 1, HIDDEN), lambda s, g: (0, 0, 0)),
                pl.BlockSpec((2, HIDDEN, D_MODEL), lambda s, g: (0, 0, 0)),
                pl.BlockSpec((2, 1, D_MODEL), lambda s, g: (0, 0, 0)),
            ],
            out_specs=pl.BlockSpec((TILE_S, BZ, D_MODEL), lambda s, g: (s, 0, 0)),
        ),
        out_shape=jax.ShapeDtypeStruct(x.shape, x.dtype),
        compiler_params=pltpu.CompilerParams(
            dimension_semantics=("arbitrary",),
        ),
    )(gid, x, w1, b1, w2, b2)


# speedup vs baseline: 2.9526x; 1.0343x over previous
import jax
import jax.numpy as jnp
from jax.experimental import pallas as pl
from jax.experimental.pallas import tpu as pltpu

DICT_LEN = 128
SEQ_LEN, BZ, D_MODEL, HIDDEN = 8192, 4, 768, 256
TILE_S = 1024


def _switcher_kernel(gid_ref, x_ref, w1_ref, b1_ref, w2_ref, b2_ref, o_ref):
    for b in range(BZ):
        g = gid_ref[b]
        x = x_ref[:, b, :]
        h = jnp.dot(x, w1_ref[g], preferred_element_type=jnp.float32)
        h = h + b1_ref[g]
        h = 0.5 * h * (1.0 + jax.lax.erf(h * 0.7071067811865476))
        out = jnp.dot(h, w2_ref[g], preferred_element_type=jnp.float32)
        o_ref[:, b, :] = out + b2_ref[g]


def kernel(x, lang_ids, w1_0, b1_0, w2_0, b2_0, w1_1, b1_1, w2_1, b2_1):
    gid = (DICT_LEN - 1 - lang_ids.astype(jnp.int32) <= 3).astype(jnp.int32)
    w1 = jnp.stack([w1_0.T, w1_1.T])
    b1 = jnp.stack([b1_0, b1_1])[:, None, :]
    w2 = jnp.stack([w2_0.T, w2_1.T])
    b2 = jnp.stack([b2_0, b2_1])[:, None, :]
    grid = (SEQ_LEN // TILE_S,)
    return pl.pallas_call(
        _switcher_kernel,
        grid_spec=pltpu.PrefetchScalarGridSpec(
            num_scalar_prefetch=1,
            grid=grid,
            in_specs=[
                pl.BlockSpec((TILE_S, BZ, D_MODEL), lambda s, g: (s, 0, 0)),
                pl.BlockSpec((2, D_MODEL, HIDDEN), lambda s, g: (0, 0, 0)),
                pl.BlockSpec((2, 1, HIDDEN), lambda s, g: (0, 0, 0)),
                pl.BlockSpec((2, HIDDEN, D_MODEL), lambda s, g: (0, 0, 0)),
                pl.BlockSpec((2, 1, D_MODEL), lambda s, g: (0, 0, 0)),
            ],
            out_specs=pl.BlockSpec((TILE_S, BZ, D_MODEL), lambda s, g: (s, 0, 0)),
        ),
        out_shape=jax.ShapeDtypeStruct(x.shape, x.dtype),
        compiler_params=pltpu.CompilerParams(
            dimension_semantics=("arbitrary",),
        ),
    )(gid, x, w1, b1, w2, b2)
